# baseline (speedup 1.0000x reference)
import math

import jax
import jax.numpy as jnp
import numpy as np
from jax.experimental import pallas as pl
from jax.experimental.pallas import tpu as pltpu


_ROTS = ((13, 15, 26, 6), (17, 29, 16, 24))


def _np_threefry2x32(k0, k1, x0, x1):
    k0 = np.uint32(k0)
    k1 = np.uint32(k1)
    x0 = np.asarray(x0, np.uint32)
    x1 = np.asarray(x1, np.uint32)
    ks = [k0, k1, np.uint32(k0 ^ k1 ^ np.uint32(0x1BD11BDA))]
    x0 = x0 + ks[0]
    x1 = x1 + ks[1]
    for i in range(5):
        for r in _ROTS[i % 2]:
            x0 = x0 + x1
            x1 = (x1 << np.uint32(r)) | (x1 >> np.uint32(32 - r))
            x1 = x1 ^ x0
        x0 = x0 + ks[(i + 1) % 3]
        x1 = x1 + ks[(i + 2) % 3] + np.uint32(i + 1)
    return x0, x1


def _subkeys():
    cnt = np.arange(6, dtype=np.uint32)
    o0, o1 = _np_threefry2x32(0, 42, np.zeros(6, np.uint32), cnt)
    return np.stack([o0, o1], axis=1)


_SK = _subkeys()
_KB, _KC, _KG, _KA, _KB2, _KCON = (tuple(int(v) for v in row) for row in _SK)

_TINY = np.float32(np.finfo(np.float32).tiny)
_NLO = np.float32(np.nextafter(np.float32(-1.0), np.float32(0.0)))
_NSPAN = np.float32(np.float32(1.0) - _NLO)
_SQRT2 = np.float32(np.sqrt(2.0))
_LOGEPS = np.float32(1e-30)
_NEG_BIG = np.float32(-1e30)


def _schedule():
    steps = 1001
    t = jnp.linspace(0.0, 1.0, steps)
    cum_prec = jnp.cos((t + 0.008) * 0.5 * math.pi / (1 + 0.008)) ** 2 * 1.00015543316
    cum_var = 1.0 - cum_prec
    sqrt_cum_prec = jnp.sqrt(cum_prec)
    sqrt_cum_var = jnp.sqrt(jnp.maximum(cum_var, 0.0))
    return sqrt_cum_prec, sqrt_cum_var


def _tf2x32(k0, k1, ks2, x1_in):
    x0 = k0
    x1 = x1_in + k1
    ks = (k0, k1, ks2)
    for i in range(5):
        for r in _ROTS[i % 2]:
            x0 = x0 + x1
            x1 = (x1 << np.uint32(r)) | (x1 >> np.uint32(32 - r))
            x1 = x1 ^ x0
        x0 = x0 + ks[(i + 1) % 3]
        x1 = x1 + ks[(i + 2) % 3] + np.uint32(i + 1)
    return x0 ^ x1


def _u01(bits):
    fb = (bits >> np.uint32(9)) | np.uint32(0x3F800000)
    return jax.lax.bitcast_convert_type(fb, jnp.float32) - np.float32(1.0)


def _gumbel_from_u01(u01):
    u = jnp.maximum(_TINY, u01 + _TINY)
    return -jnp.log(-jnp.log(u))


def _normal_from_u01(u01):
    u = jnp.maximum(_NLO, u01 * _NSPAN + _NLO)
    return _SQRT2 * jax.lax.erf_inv(u)


def _idiv3(q):
    return jnp.floor(q.astype(jnp.float32) * np.float32(1.0 / 3.0)).astype(jnp.int32)


def _bf(v):
    return v.astype(jnp.bfloat16).astype(jnp.float32)


_EDGE_S = 384


def _edges_kernel(bp_ref, x_ref, o_ref):
    S = x_ref.shape[0]
    x = x_ref[...]
    bp = bp_ref[...]

    si = jax.lax.broadcasted_iota(jnp.int32, (S, 128), 0)
    lane = jax.lax.broadcasted_iota(jnp.int32, (S, 128), 1)
    p = (pl.program_id(0) * S + si) * 128 + lane

    q = p >> 2
    t3 = _idiv3(q)
    ig = (t3 << 2) | (p & 3)
    g = q - 3 * t3

    def sel(vals):
        v0, v1, v2 = (np.uint32(v) for v in vals)
        return jnp.where(g == 0, v0, jnp.where(g == 1, v1, v2)).astype(jnp.uint32)

    k0 = sel((_KA[0], _KB2[0], _KCON[0]))
    k1 = sel((_KA[1], _KB2[1], _KCON[1]))
    ks2 = k0 ^ k1 ^ np.uint32(0x1BD11BDA)

    bits = _tf2x32(k0, k1, ks2, ig.astype(jnp.uint32))
    gum = _gumbel_from_u01(_u01(bits))

    l2 = lane & 1
    l4 = lane & 3

    def bfly(v, combine):
        a = combine(v, jnp.where(l2 == 0, jnp.roll(v, -1, axis=1), jnp.roll(v, 1, axis=1)))
        return combine(a, jnp.where(l4 < 2, jnp.roll(a, -2, axis=1), jnp.roll(a, 2, axis=1)))

    xb = _bf(x)
    ob = _bf((np.float32(1.0) - bp) * np.float32(0.25))
    db = _bf(bp + (np.float32(1.0) - bp) * np.float32(0.25))
    rolls = {s: (xb if s == 0 else jnp.roll(xb, s, axis=1))
             for s in (-3, -2, -1, 0, 1, 2, 3)}

    def bcast(j):
        return jnp.where(l4 == 0, rolls[-j],
                         jnp.where(l4 == 1, rolls[1 - j],
                                   jnp.where(l4 == 2, rolls[2 - j], rolls[3 - j])))

    acc = None
    for j in range(4):
        t = bcast(j) * jnp.where(l4 == j, db, ob)
        acc = t if acc is None else acc + t
    y = jnp.log(jnp.maximum(acc, _LOGEPS)) + gum

    m4 = bfly(y, jnp.maximum)
    cand = jnp.where(y == m4, l4, 4)
    cmin = bfly(cand, jnp.minimum)
    o_ref[...] = (l4 == cmin).astype(jnp.float32)


_NODE_S = 1024


def _nodes_kernel(x_ref, bp_ref, bv_ref, ob5_ref, db5_ref, on_ref, og_ref):
    S = x_ref.shape[0]
    x = x_ref[...]
    bp = bp_ref[...]
    bv = bv_ref[...]
    ob5 = ob5_ref[...]
    db5 = db5_ref[...]

    r = pl.program_id(0) * S + jax.lax.broadcasted_iota(jnp.int32, (S, 13), 0)
    c = jax.lax.broadcasted_iota(jnp.int32, (S, 13), 1)

    is_flag = c == 0
    is_cat = jnp.logical_and(c >= 1, c <= 5)

    i1 = jnp.where(is_flag, 2 * r,
                   jnp.where(is_cat, 5 * r + (c - 1), 7 * r + (c - 6)))

    def sel(vals):
        v0, v1, v2 = (np.uint32(v) for v in vals)
        return jnp.where(is_flag, v0, jnp.where(is_cat, v1, v2)).astype(jnp.uint32)

    k0 = sel((_KB[0], _KC[0], _KG[0]))
    k1 = sel((_KB[1], _KC[1], _KG[1]))
    ks2 = k0 ^ k1 ^ np.uint32(0x1BD11BDA)
    u1 = _u01(_tf2x32(k0, k1, ks2, i1.astype(jnp.uint32)))

    kb0 = jnp.full((S, 13), _KB[0], jnp.uint32)
    kb1 = jnp.full((S, 13), _KB[1], jnp.uint32)
    kbs2 = kb0 ^ kb1 ^ np.uint32(0x1BD11BDA)
    u2 = _u01(_tf2x32(kb0, kb1, kbs2, (2 * r + 1).astype(jnp.uint32)))

    gum1 = _gumbel_from_u01(u1)
    gum2 = _gumbel_from_u01(u2)
    nrm = _normal_from_u01(u1)

    one = np.float32(1.0)

    ob2 = _bf((one - bp) * np.float32(0.5))
    db2 = _bf(bp + (one - bp) * np.float32(0.5))
    xb0 = _bf(one - x)
    xb1 = _bf(x)
    y0 = jnp.log(jnp.maximum(xb0 * db2 + xb1 * ob2, _LOGEPS)) + gum1
    y1 = jnp.log(jnp.maximum(xb0 * ob2 + xb1 * db2, _LOGEPS)) + gum2
    flag_out = (y1 > y0).astype(jnp.float32)

    xcb = _bf(x)
    acc = None
    for j in range(1, 6):
        t = xcb[:, j:j + 1] * jnp.where(c == j, db5, ob5)
        acc = t if acc is None else acc + t
    y5 = jnp.log(jnp.maximum(acc, _LOGEPS)) + gum1
    y5m = jnp.where(is_cat, y5, _NEG_BIG)
    m5 = jnp.max(y5m, axis=1, keepdims=True)
    cand = jnp.where(y5m == m5, c, 99)
    cmin = jnp.min(cand, axis=1, keepdims=True)
    cat_out = (c == cmin).astype(jnp.float32)

    gauss_out = bp * x + bv * nrm

    on_ref[...] = jnp.where(is_flag, flag_out,
                            jnp.where(is_cat, cat_out, gauss_out))
    og_ref[...] = nrm[:, 6:13]


def kernel(nodes, edges, timestep):
    sqrt_cum_prec, sqrt_cum_var = _schedule()
    bp = sqrt_cum_prec[timestep]
    bv = sqrt_cum_var[timestep]

    ef = edges.reshape(49152, 128)
    bp_e = jnp.repeat(bp, 384)[:, None]
    grid_e = 49152 // _EDGE_S
    noisy_edges = pl.pallas_call(
        _edges_kernel,
        grid=(grid_e,),
        in_specs=[
            pl.BlockSpec((_EDGE_S, 1), lambda i: (i, 0)),
            pl.BlockSpec((_EDGE_S, 128), lambda i: (i, 0)),
        ],
        out_specs=pl.BlockSpec((_EDGE_S, 128), lambda i: (i, 0)),
        out_shape=jax.ShapeDtypeStruct((49152, 128), jnp.float32),
    )(bp_e, ef).reshape(128, 64, 64, 12)

    nf = nodes.reshape(8192, 13)
    bp_n = jnp.repeat(bp, 64)[:, None]
    bv_n = jnp.repeat(bv, 64)[:, None]
    ob5_n = ((1.0 - bp_n) / 5.0).astype(jnp.bfloat16).astype(jnp.float32)
    db5_n = (bp_n + (1.0 - bp_n) / 5.0).astype(jnp.bfloat16).astype(jnp.float32)
    grid_n = 8192 // _NODE_S
    noisy_nodes, gnoise = pl.pallas_call(
        _nodes_kernel,
        grid=(grid_n,),
        in_specs=[
            pl.BlockSpec((_NODE_S, 13), lambda i: (i, 0)),
            pl.BlockSpec((_NODE_S, 1), lambda i: (i, 0)),
            pl.BlockSpec((_NODE_S, 1), lambda i: (i, 0)),
            pl.BlockSpec((_NODE_S, 1), lambda i: (i, 0)),
            pl.BlockSpec((_NODE_S, 1), lambda i: (i, 0)),
        ],
        out_specs=[
            pl.BlockSpec((_NODE_S, 13), lambda i: (i, 0)),
            pl.BlockSpec((_NODE_S, 7), lambda i: (i, 0)),
        ],
        out_shape=[
            jax.ShapeDtypeStruct((8192, 13), jnp.float32),
            jax.ShapeDtypeStruct((8192, 7), jnp.float32),
        ],
    )(nf, bp_n, bv_n, ob5_n, db5_n)

    return (noisy_nodes.reshape(128, 64, 13),
            noisy_edges,
            gnoise.reshape(128, 64, 7))


# speedup vs baseline: 1.4253x; 1.4253x over previous
import math

import jax
import jax.numpy as jnp
import numpy as np
from jax.experimental import pallas as pl
from jax.experimental.pallas import tpu as pltpu


_ROTS = ((13, 15, 26, 6), (17, 29, 16, 24))


def _np_threefry2x32(k0, k1, x0, x1):
    k0 = np.uint32(k0)
    k1 = np.uint32(k1)
    x0 = np.asarray(x0, np.uint32)
    x1 = np.asarray(x1, np.uint32)
    ks = [k0, k1, np.uint32(k0 ^ k1 ^ np.uint32(0x1BD11BDA))]
    x0 = x0 + ks[0]
    x1 = x1 + ks[1]
    for i in range(5):
        for r in _ROTS[i % 2]:
            x0 = x0 + x1
            x1 = (x1 << np.uint32(r)) | (x1 >> np.uint32(32 - r))
            x1 = x1 ^ x0
        x0 = x0 + ks[(i + 1) % 3]
        x1 = x1 + ks[(i + 2) % 3] + np.uint32(i + 1)
    return x0, x1


def _subkeys():
    cnt = np.arange(6, dtype=np.uint32)
    o0, o1 = _np_threefry2x32(0, 42, np.zeros(6, np.uint32), cnt)
    return np.stack([o0, o1], axis=1)


_SK = _subkeys()
_KB, _KC, _KG, _KA, _KB2, _KCON = (tuple(int(v) for v in row) for row in _SK)

_TINY = np.float32(np.finfo(np.float32).tiny)
_NLO = np.float32(np.nextafter(np.float32(-1.0), np.float32(0.0)))
_NSPAN = np.float32(np.float32(1.0) - _NLO)
_SQRT2 = np.float32(np.sqrt(2.0))
_LOGEPS = np.float32(1e-30)
_NEG_BIG = np.float32(-1e30)


def _schedule():
    steps = 1001
    t = jnp.linspace(0.0, 1.0, steps)
    cum_prec = jnp.cos((t + 0.008) * 0.5 * math.pi / (1 + 0.008)) ** 2 * 1.00015543316
    cum_var = 1.0 - cum_prec
    sqrt_cum_prec = jnp.sqrt(cum_prec)
    sqrt_cum_var = jnp.sqrt(jnp.maximum(cum_var, 0.0))
    return sqrt_cum_prec, sqrt_cum_var


def _tf2x32(k0, k1, ks2, x1_in):
    x0 = k0
    x1 = x1_in + k1
    ks = (k0, k1, ks2)
    for i in range(5):
        for r in _ROTS[i % 2]:
            x0 = x0 + x1
            x1 = (x1 << np.uint32(r)) | (x1 >> np.uint32(32 - r))
            x1 = x1 ^ x0
        x0 = x0 + ks[(i + 1) % 3]
        x1 = x1 + ks[(i + 2) % 3] + np.uint32(i + 1)
    return x0 ^ x1


def _u01(bits):
    fb = (bits >> np.uint32(9)) | np.uint32(0x3F800000)
    return jax.lax.bitcast_convert_type(fb, jnp.float32) - np.float32(1.0)


def _gumbel_from_u01(u01):
    u = jnp.maximum(_TINY, u01 + _TINY)
    return -jnp.log(-jnp.log(u))


def _normal_from_u01(u01):
    u = jnp.maximum(_NLO, u01 * _NSPAN + _NLO)
    return _SQRT2 * jax.lax.erf_inv(u)


def _bf(v):
    return v.astype(jnp.bfloat16).astype(jnp.float32)


_EDGE_N = 1024


def _edges_kernel(bp_ref, x_ref, o_ref):
    N = x_ref.shape[0]
    xt = x_ref[...].T
    bp = bp_ref[...]

    c12 = jax.lax.broadcasted_iota(jnp.int32, (12, N), 0)
    r = pl.program_id(0) * N + jax.lax.broadcasted_iota(jnp.int32, (12, N), 1)
    c4 = c12 & 3
    ig = (r << 2) | c4

    def sel(vals):
        v0, v1, v2 = (np.uint32(v) for v in vals)
        return jnp.where(c12 < 4, v0, jnp.where(c12 < 8, v1, v2)).astype(jnp.uint32)

    k0 = sel((_KA[0], _KB2[0], _KCON[0]))
    k1 = sel((_KA[1], _KB2[1], _KCON[1]))
    ks2 = k0 ^ k1 ^ np.uint32(0x1BD11BDA)

    bits = _tf2x32(k0, k1, ks2, ig.astype(jnp.uint32))
    gum = _gumbel_from_u01(_u01(bits))

    xb = _bf(xt)
    ob = _bf((np.float32(1.0) - bp) * np.float32(0.25))
    db = _bf(bp + (np.float32(1.0) - bp) * np.float32(0.25))
    rolls = {s: (xb if s == 0 else jnp.roll(xb, s, axis=0))
             for s in (-3, -2, -1, 0, 1, 2, 3)}

    def bcast(j):
        return jnp.where(c4 == 0, rolls[-j],
                         jnp.where(c4 == 1, rolls[1 - j],
                                   jnp.where(c4 == 2, rolls[2 - j], rolls[3 - j])))

    acc = None
    for j in range(4):
        t = bcast(j) * jnp.where(c4 == j, db, ob)
        acc = t if acc is None else acc + t
    y = jnp.log(jnp.maximum(acc, _LOGEPS)) + gum

    l2 = c12 & 1

    def bfly(v, combine):
        a = combine(v, jnp.where(l2 == 0, jnp.roll(v, -1, axis=0), jnp.roll(v, 1, axis=0)))
        return combine(a, jnp.where(c4 < 2, jnp.roll(a, -2, axis=0), jnp.roll(a, 2, axis=0)))

    m4 = bfly(y, jnp.maximum)
    cand = jnp.where(y == m4, c4, 4)
    cmin = bfly(cand, jnp.minimum)
    o_ref[...] = ((c4 == cmin).astype(jnp.float32)).T


_NODE_N = 1024


def _nodes_kernel(x_ref, bp_ref, bv_ref, ob5_ref, db5_ref, on_ref, og_ref):
    N = x_ref.shape[0]
    xt = x_ref[...].T
    bp = bp_ref[...]
    bv = bv_ref[...]
    ob5 = ob5_ref[...]
    db5 = db5_ref[...]

    c = jax.lax.broadcasted_iota(jnp.int32, (14, N), 0)
    r = pl.program_id(0) * N + jax.lax.broadcasted_iota(jnp.int32, (14, N), 1)

    is_flag = c == 0
    is_cat = jnp.logical_and(c >= 1, c <= 5)
    is_g = jnp.logical_and(c >= 6, c <= 12)

    i1 = jnp.where(is_flag, 2 * r,
                   jnp.where(is_cat, 5 * r + (c - 1),
                             jnp.where(is_g, 7 * r + (c - 6), 2 * r + 1)))

    def sel(vals):
        v0, v1, v2, v3 = (np.uint32(v) for v in vals)
        return jnp.where(is_flag, v0,
                         jnp.where(is_cat, v1,
                                   jnp.where(is_g, v2, v3))).astype(jnp.uint32)

    k0 = sel((_KB[0], _KC[0], _KG[0], _KB[0]))
    k1 = sel((_KB[1], _KC[1], _KG[1], _KB[1]))
    ks2 = k0 ^ k1 ^ np.uint32(0x1BD11BDA)
    u = _u01(_tf2x32(k0, k1, ks2, i1.astype(jnp.uint32)))

    gum = _gumbel_from_u01(u[0:13])
    gum2 = _gumbel_from_u01(u[13:14])
    nrm = _normal_from_u01(u[6:13])

    one = np.float32(1.0)
    c13 = c[0:13]

    f = xt[0:1]
    ob2 = _bf((one - bp) * np.float32(0.5))
    db2 = _bf(bp + (one - bp) * np.float32(0.5))
    fb0 = _bf(one - f)
    fb1 = _bf(f)
    y0 = jnp.log(jnp.maximum(fb0 * db2 + fb1 * ob2, _LOGEPS)) + gum[0:1]
    y1 = jnp.log(jnp.maximum(fb0 * ob2 + fb1 * db2, _LOGEPS)) + gum2
    flag_out = (y1 > y0).astype(jnp.float32)

    xb = _bf(xt)
    acc = None
    for j in range(1, 6):
        t = xb[j:j + 1] * jnp.where(c13 == j, db5, ob5)
        acc = t if acc is None else acc + t
    y5 = jnp.log(jnp.maximum(acc, _LOGEPS)) + gum
    y5m = jnp.where(jnp.logical_and(c13 >= 1, c13 <= 5), y5, _NEG_BIG)
    m5 = jnp.max(y5m, axis=0, keepdims=True)
    cand = jnp.where(y5m == m5, c13, 99)
    cmin = jnp.min(cand, axis=0, keepdims=True)
    cat_out = (c13 == cmin).astype(jnp.float32)

    gauss_out = bp * xt[6:13] + bv * nrm

    out = jnp.where(c13 == 0, flag_out,
                    jnp.where(jnp.logical_and(c13 >= 1, c13 <= 5), cat_out,
                              jnp.concatenate([jnp.zeros((6, N), jnp.float32),
                                               gauss_out], axis=0)))
    on_ref[...] = out.T
    og_ref[...] = nrm.T


def kernel(nodes, edges, timestep):
    sqrt_cum_prec, sqrt_cum_var = _schedule()
    bp = sqrt_cum_prec[timestep]
    bv = sqrt_cum_var[timestep]

    ef = edges.reshape(524288, 12)
    bp_e = jnp.repeat(bp, 4096)[None, :]
    grid_e = 524288 // _EDGE_N
    noisy_edges = pl.pallas_call(
        _edges_kernel,
        grid=(grid_e,),
        in_specs=[
            pl.BlockSpec((1, _EDGE_N), lambda i: (0, i)),
            pl.BlockSpec((_EDGE_N, 12), lambda i: (i, 0)),
        ],
        out_specs=pl.BlockSpec((_EDGE_N, 12), lambda i: (i, 0)),
        out_shape=jax.ShapeDtypeStruct((524288, 12), jnp.float32),
    )(bp_e, ef).reshape(128, 64, 64, 12)

    nf = nodes.reshape(8192, 13)
    bp_n = jnp.repeat(bp, 64)[None, :]
    bv_n = jnp.repeat(bv, 64)[None, :]
    ob5_n = ((1.0 - bp_n) / 5.0).astype(jnp.bfloat16).astype(jnp.float32)
    db5_n = (bp_n + (1.0 - bp_n) / 5.0).astype(jnp.bfloat16).astype(jnp.float32)
    grid_n = 8192 // _NODE_N
    noisy_nodes, gnoise = pl.pallas_call(
        _nodes_kernel,
        grid=(grid_n,),
        in_specs=[
            pl.BlockSpec((_NODE_N, 13), lambda i: (i, 0)),
            pl.BlockSpec((1, _NODE_N), lambda i: (0, i)),
            pl.BlockSpec((1, _NODE_N), lambda i: (0, i)),
            pl.BlockSpec((1, _NODE_N), lambda i: (0, i)),
            pl.BlockSpec((1, _NODE_N), lambda i: (0, i)),
        ],
        out_specs=[
            pl.BlockSpec((_NODE_N, 13), lambda i: (i, 0)),
            pl.BlockSpec((_NODE_N, 7), lambda i: (i, 0)),
        ],
        out_shape=[
            jax.ShapeDtypeStruct((8192, 13), jnp.float32),
            jax.ShapeDtypeStruct((8192, 7), jnp.float32),
        ],
    )(nf, bp_n, bv_n, ob5_n, db5_n)

    return (noisy_nodes.reshape(128, 64, 13),
            noisy_edges,
            gnoise.reshape(128, 64, 7))


# speedup vs baseline: 6.9731x; 4.8924x over previous
import math

import jax
import jax.numpy as jnp
import numpy as np
from jax.experimental import pallas as pl
from jax.experimental.pallas import tpu as pltpu


_ROTS = ((13, 15, 26, 6), (17, 29, 16, 24))


def _np_threefry2x32(k0, k1, x0, x1):
    k0 = np.uint32(k0)
    k1 = np.uint32(k1)
    x0 = np.asarray(x0, np.uint32)
    x1 = np.asarray(x1, np.uint32)
    ks = [k0, k1, np.uint32(k0 ^ k1 ^ np.uint32(0x1BD11BDA))]
    x0 = x0 + ks[0]
    x1 = x1 + ks[1]
    for i in range(5):
        for r in _ROTS[i % 2]:
            x0 = x0 + x1
            x1 = (x1 << np.uint32(r)) | (x1 >> np.uint32(32 - r))
            x1 = x1 ^ x0
        x0 = x0 + ks[(i + 1) % 3]
        x1 = x1 + ks[(i + 2) % 3] + np.uint32(i + 1)
    return x0, x1


def _subkeys():
    cnt = np.arange(6, dtype=np.uint32)
    o0, o1 = _np_threefry2x32(0, 42, np.zeros(6, np.uint32), cnt)
    return np.stack([o0, o1], axis=1)


_SK = _subkeys()
_KB, _KC, _KG, _KA, _KB2, _KCON = (tuple(int(v) for v in row) for row in _SK)

_TINY = np.float32(np.finfo(np.float32).tiny)
_NLO = np.float32(np.nextafter(np.float32(-1.0), np.float32(0.0)))
_NSPAN = np.float32(np.float32(1.0) - _NLO)
_SQRT2 = np.float32(np.sqrt(2.0))
_LOGEPS = np.float32(1e-30)


def _schedule():
    steps = 1001
    t = jnp.linspace(0.0, 1.0, steps)
    cum_prec = jnp.cos((t + 0.008) * 0.5 * math.pi / (1 + 0.008)) ** 2 * 1.00015543316
    cum_var = 1.0 - cum_prec
    sqrt_cum_prec = jnp.sqrt(cum_prec)
    sqrt_cum_var = jnp.sqrt(jnp.maximum(cum_var, 0.0))
    return sqrt_cum_prec, sqrt_cum_var


def _tf2x32(k0, k1, ks2, x1_in):
    x0 = k0
    x1 = x1_in + k1
    ks = (k0, k1, ks2)
    for i in range(5):
        for r in _ROTS[i % 2]:
            x0 = x0 + x1
            x1 = (x1 << np.uint32(r)) | (x1 >> np.uint32(32 - r))
            x1 = x1 ^ x0
        x0 = x0 + ks[(i + 1) % 3]
        x1 = x1 + ks[(i + 2) % 3] + np.uint32(i + 1)
    return x0 ^ x1


def _u01(bits):
    fb = (bits >> np.uint32(9)) | np.uint32(0x3F800000)
    return jax.lax.bitcast_convert_type(fb, jnp.float32) - np.float32(1.0)


def _gumbel_from_u01(u01):
    u = jnp.maximum(_TINY, u01 + _TINY)
    return -jnp.log(-jnp.log(u))


def _normal_from_u01(u01):
    u = jnp.maximum(_NLO, u01 * _NSPAN + _NLO)
    return _SQRT2 * jax.lax.erf_inv(u)


def _bf(v):
    return v.astype(jnp.bfloat16).astype(jnp.float32)


def _onehot_argmax4(ys):
    m = jnp.maximum(jnp.maximum(ys[0], ys[1]), jnp.maximum(ys[2], ys[3]))
    cand = [jnp.where(ys[c] == m, np.int32(c), np.int32(4)) for c in range(4)]
    cmin = jnp.minimum(jnp.minimum(cand[0], cand[1]),
                       jnp.minimum(cand[2], cand[3]))
    return [(cand[c] == cmin).astype(jnp.float32) for c in range(4)]


def _edges_kernel(bp_ref, x_ref, o_ref):
    x = x_ref[...]
    bp = bp_ref[...]

    row = jax.lax.broadcasted_iota(jnp.int32, (768, 128), 0)
    lane = jax.lax.broadcasted_iota(jnp.int32, (768, 128), 1)
    j = row & 63
    ch = row >> 6
    c4 = ch & 3
    rlog = (lane << 12) + pl.program_id(0) * 64 + j
    ig = (rlog << 2) | c4

    def sel(vals):
        v0, v1, v2 = (np.uint32(v) for v in vals)
        return jnp.where(ch < 4, v0, jnp.where(ch < 8, v1, v2)).astype(jnp.uint32)

    k0 = sel((_KA[0], _KB2[0], _KCON[0]))
    k1 = sel((_KA[1], _KB2[1], _KCON[1]))
    ks2 = k0 ^ k1 ^ np.uint32(0x1BD11BDA)

    gum = _gumbel_from_u01(_u01(_tf2x32(k0, k1, ks2, ig.astype(jnp.uint32))))

    xb = _bf(x)
    ob = _bf((np.float32(1.0) - bp) * np.float32(0.25))
    db = _bf(bp + (np.float32(1.0) - bp) * np.float32(0.25))

    for g in range(3):
        xg = [xb[(4 * g + c) * 64:(4 * g + c + 1) * 64] for c in range(4)]
        ys = []
        for c in range(4):
            acc = None
            for jj in range(4):
                t = xg[jj] * (db if jj == c else ob)
                acc = t if acc is None else acc + t
            sl = slice((4 * g + c) * 64, (4 * g + c + 1) * 64)
            ys.append(jnp.log(jnp.maximum(acc, _LOGEPS)) + gum[sl])
        oh = _onehot_argmax4(ys)
        for c in range(4):
            o_ref[(4 * g + c) * 64:(4 * g + c + 1) * 64, :] = oh[c]


def _nodes_kernel(x_ref, bp_ref, bv_ref, ob5_ref, db5_ref, on_ref, og_ref):
    x = x_ref[...]
    bp = bp_ref[...]
    bv = bv_ref[...]
    ob5 = ob5_ref[...]
    db5 = db5_ref[...]

    row = jax.lax.broadcasted_iota(jnp.int32, (896, 128), 0)
    lane = jax.lax.broadcasted_iota(jnp.int32, (896, 128), 1)
    n = row & 63
    ch = row >> 6
    rn = (lane << 6) + n

    is_flag = ch == 0
    is_cat = jnp.logical_and(ch >= 1, ch <= 5)
    is_g = jnp.logical_and(ch >= 6, ch <= 12)

    i1 = jnp.where(is_flag, 2 * rn,
                   jnp.where(is_cat, 5 * rn + (ch - 1),
                             jnp.where(is_g, 7 * rn + (ch - 6), 2 * rn + 1)))

    def sel(vals):
        v0, v1, v2, v3 = (np.uint32(v) for v in vals)
        return jnp.where(is_flag, v0,
                         jnp.where(is_cat, v1,
                                   jnp.where(is_g, v2, v3))).astype(jnp.uint32)

    k0 = sel((_KB[0], _KC[0], _KG[0], _KB[0]))
    k1 = sel((_KB[1], _KC[1], _KG[1], _KB[1]))
    ks2 = k0 ^ k1 ^ np.uint32(0x1BD11BDA)
    u = _u01(_tf2x32(k0, k1, ks2, i1.astype(jnp.uint32)))

    one = np.float32(1.0)

    f = x[0:64]
    gum0 = _gumbel_from_u01(u[0:64])
    gum2 = _gumbel_from_u01(u[832:896])
    ob2 = _bf((one - bp) * np.float32(0.5))
    db2 = _bf(bp + (one - bp) * np.float32(0.5))
    fb0 = _bf(one - f)
    fb1 = _bf(f)
    y0 = jnp.log(jnp.maximum(fb0 * db2 + fb1 * ob2, _LOGEPS)) + gum0
    y1 = jnp.log(jnp.maximum(fb0 * ob2 + fb1 * db2, _LOGEPS)) + gum2
    on_ref[0:64, :] = (y1 > y0).astype(jnp.float32)

    xc = [_bf(x[jj * 64:(jj + 1) * 64]) for jj in range(1, 6)]
    ys = []
    for c in range(5):
        acc = None
        for jj in range(5):
            t = xc[jj] * (db5 if jj == c else ob5)
            acc = t if acc is None else acc + t
        sl = slice((c + 1) * 64, (c + 2) * 64)
        ys.append(jnp.log(jnp.maximum(acc, _LOGEPS)) + _gumbel_from_u01(u[sl]))
    m = jnp.maximum(jnp.maximum(jnp.maximum(ys[0], ys[1]),
                                jnp.maximum(ys[2], ys[3])), ys[4])
    cand = [jnp.where(ys[c] == m, np.int32(c), np.int32(5)) for c in range(5)]
    cmin = jnp.minimum(jnp.minimum(jnp.minimum(cand[0], cand[1]),
                                   jnp.minimum(cand[2], cand[3])), cand[4])
    for c in range(5):
        on_ref[(c + 1) * 64:(c + 2) * 64, :] = (cand[c] == cmin).astype(jnp.float32)

    nrm = _normal_from_u01(u[384:832])
    on_ref[384:832, :] = bp * x[384:832] + bv * nrm
    og_ref[...] = nrm


def kernel(nodes, edges, timestep):
    sqrt_cum_prec, sqrt_cum_var = _schedule()
    bp = sqrt_cum_prec[timestep][None, :]
    bv = sqrt_cum_var[timestep][None, :]
    ob5 = ((1.0 - bp) / 5.0).astype(jnp.bfloat16).astype(jnp.float32)
    db5 = (bp + (1.0 - bp) / 5.0).astype(jnp.bfloat16).astype(jnp.float32)

    ev = edges.transpose(1, 3, 2, 0).reshape(49152, 128)
    noisy_edges = pl.pallas_call(
        _edges_kernel,
        grid=(64,),
        in_specs=[
            pl.BlockSpec((1, 128), lambda i: (0, 0)),
            pl.BlockSpec((768, 128), lambda i: (i, 0)),
        ],
        out_specs=pl.BlockSpec((768, 128), lambda i: (i, 0)),
        out_shape=jax.ShapeDtypeStruct((49152, 128), jnp.float32),
    )(bp, ev).reshape(64, 12, 64, 128).transpose(3, 0, 2, 1)

    nv = nodes.transpose(2, 1, 0).reshape(832, 128)
    noisy_nodes, gnoise = pl.pallas_call(
        _nodes_kernel,
        grid=(1,),
        in_specs=[
            pl.BlockSpec((832, 128), lambda i: (0, 0)),
            pl.BlockSpec((1, 128), lambda i: (0, 0)),
            pl.BlockSpec((1, 128), lambda i: (0, 0)),
            pl.BlockSpec((1, 128), lambda i: (0, 0)),
            pl.BlockSpec((1, 128), lambda i: (0, 0)),
        ],
        out_specs=[
            pl.BlockSpec((832, 128), lambda i: (0, 0)),
            pl.BlockSpec((448, 128), lambda i: (0, 0)),
        ],
        out_shape=[
            jax.ShapeDtypeStruct((832, 128), jnp.float32),
            jax.ShapeDtypeStruct((448, 128), jnp.float32),
        ],
    )(nv, bp, bv, ob5, db5)

    return (noisy_nodes.reshape(13, 64, 128).transpose(2, 1, 0),
            noisy_edges,
            gnoise.reshape(7, 64, 128).transpose(2, 1, 0))
